# Initial kernel scaffold; baseline (speedup 1.0000x reference)
import jax
import jax.numpy as jnp
from jax.experimental import pallas as pl


def kernel(x, edge_index, pairs, W1l, b1l, W1r, W2l, b2l, W2r, Wlp, blp):
    raise NotImplementedError("write your pallas kernel here")


# speedup vs baseline: 5.0686x; 5.0686x over previous
import jax
import jax.numpy as jnp
from jax import lax
from jax.experimental import pallas as pl
from jax.experimental.pallas import tpu as pltpu
from jax.experimental.pallas import tpu_sc as plsc

N = 10000
NPAD = 10240
E = 320000
D = 128
P = 4096

NC = 2
NS = 16
NW = NC * NS
L = 16

EPT = E // NW
CH = 80
NCHUNK = EPT // CH
RPT = NPAD // NS
PPT = P // NW
BR = 2048

_F32 = jnp.float32


def _mk_mesh():
    return plsc.VectorSubcoreMesh(
        core_axis_name="c", subcore_axis_name="s", num_cores=NC, num_subcores=NS
    )


def _agg_body(with_counts):
    def body(feat, src, dst, zeros, *rest):
        if with_counts:
            psum, cnts, acc, src_v, dst_v, rows_v, cnt_v, sem = rest
        else:
            psum, acc, src_v, dst_v, rows_v, sem = rest
        c = lax.axis_index("c")
        s = lax.axis_index("s")
        wid = s * NC + c

        if with_counts:
            z16 = jnp.zeros((L,), _F32)

            def zloop(i, carry):
                cnt_v[pl.ds(i * L, L)] = z16
                return carry

            lax.fori_loop(0, NPAD // L, zloop, 0)

        pltpu.sync_copy(zeros.at[pl.ds(s * RPT, RPT)], acc.at[pl.ds(s * RPT, RPT)])
        plsc.subcore_barrier()

        ones16 = jnp.ones((L,), _F32)

        def chunk(i, carry):
            base = wid * EPT + i * CH
            pltpu.sync_copy(src.at[pl.ds(base, CH)], src_v)
            pltpu.sync_copy(dst.at[pl.ds(base, CH)], dst_v)
            pltpu.async_copy(feat.at[src_v], rows_v, sem).wait()
            pltpu.sync_copy(rows_v, acc.at[dst_v], add=True)
            if with_counts:
                for j in range(CH // L):
                    idx = dst_v[pl.ds(j * L, L)]
                    plsc.addupdate_scatter(cnt_v, [idx], ones16)
            return carry

        lax.fori_loop(0, NCHUNK, chunk, 0)
        if with_counts:
            pltpu.sync_copy(cnt_v, cnts.at[wid])
        plsc.subcore_barrier()

        pltpu.sync_copy(acc.at[pl.ds(s * RPT, RPT)], psum.at[c, pl.ds(s * RPT, RPT)])

    return body


def _make_agg(with_counts):
    out_type = [jax.ShapeDtypeStruct((NC, NPAD, D), _F32)]
    if with_counts:
        out_type.append(jax.ShapeDtypeStruct((NW, NPAD), _F32))
    scratch = [
        pltpu.VMEM_SHARED((NPAD, D), _F32),
        pltpu.VMEM((CH,), jnp.int32),
        pltpu.VMEM((CH,), jnp.int32),
        pltpu.VMEM((CH, D), _F32),
    ]
    if with_counts:
        scratch.append(pltpu.VMEM((NPAD,), _F32))
    scratch.append(pltpu.SemaphoreType.DMA)
    return pl.kernel(
        _agg_body(with_counts),
        out_type=out_type,
        mesh=_mk_mesh(),
        compiler_params=pltpu.CompilerParams(needs_layout_passes=False),
        scratch_types=scratch,
    )


def _tc_layer1(ps, cb, ft, wl, wr, bl, oh, orc):
    ssum = ps[0] + ps[1]
    cnt = jnp.sum(cb[...], axis=0).reshape(BR, 1)
    recip = 1.0 / jnp.maximum(cnt, 1.0)
    mean = ssum * recip
    acc = lax.dot_general(mean, wl[...], (((1,), (1,)), ((), ())),
                          preferred_element_type=_F32)
    acc += lax.dot_general(ft[...], wr[...], (((1,), (1,)), ((), ())),
                           preferred_element_type=_F32)
    acc += bl[...]
    oh[...] = jnp.maximum(acc, 0.0)
    orc[...] = jnp.broadcast_to(recip, (BR, 8))


def _tc_layer2(ps, rc, ft, wl, wr, bl, wuv, buv, o):
    mean = (ps[0] + ps[1]) * rc[:, :1]
    h2 = lax.dot_general(mean, wl[...], (((1,), (1,)), ((), ())),
                         preferred_element_type=_F32)
    h2 += lax.dot_general(ft[...], wr[...], (((1,), (1,)), ((), ())),
                          preferred_element_type=_F32)
    h2 += bl[...]
    o[...] = lax.dot_general(h2, wuv[...], (((1,), (0,)), ((), ())),
                             preferred_element_type=_F32) + buv[...]


def _pairs_body(uh, vh, p0, p1, out, u_v, v_v, p0_v, p1_v, out_v):
    c = lax.axis_index("c")
    s = lax.axis_index("s")
    wid = s * NC + c
    pltpu.sync_copy(uh, u_v)
    pltpu.sync_copy(vh, v_v)
    pltpu.sync_copy(p0.at[pl.ds(wid * PPT, PPT)], p0_v)
    pltpu.sync_copy(p1.at[pl.ds(wid * PPT, PPT)], p1_v)
    for j in range(PPT // L):
        i0 = p0_v[pl.ds(j * L, L)]
        i1 = p1_v[pl.ds(j * L, L)]
        u = plsc.load_gather(u_v, [i0])
        v = plsc.load_gather(v_v, [i1])
        z = u + v
        out_v[pl.ds(j * L, L)] = 1.0 / (1.0 + jnp.exp(-z))
    pltpu.sync_copy(out_v, out.at[pl.ds(wid * PPT, PPT)])


def kernel(x, edge_index, pairs, W1l, b1l, W1r, W2l, b2l, W2r, Wlp, blp):
    src = edge_index[0].astype(jnp.int32)
    dst = edge_index[1].astype(jnp.int32)
    p0 = pairs[:, 0].astype(jnp.int32)
    p1 = pairs[:, 1].astype(jnp.int32)

    xp = jnp.pad(x, ((0, NPAD - N), (0, 0)))
    zeros_d = jnp.zeros((NPAD, D), _F32)

    psum1, cnts = _make_agg(True)(xp, src, dst, zeros_d)

    wspec = pl.BlockSpec((D, D), lambda i: (0, 0))
    bspec = pl.BlockSpec((1, D), lambda i: (0, 0))
    h, rec = pl.pallas_call(
        _tc_layer1,
        grid=(NPAD // BR,),
        in_specs=[
            pl.BlockSpec((NC, BR, D), lambda i: (0, i, 0)),
            pl.BlockSpec((NW, BR), lambda i: (0, i)),
            pl.BlockSpec((BR, D), lambda i: (i, 0)),
            wspec, wspec, bspec,
        ],
        out_specs=[
            pl.BlockSpec((BR, D), lambda i: (i, 0)),
            pl.BlockSpec((BR, 8), lambda i: (i, 0)),
        ],
        out_shape=[
            jax.ShapeDtypeStruct((NPAD, D), _F32),
            jax.ShapeDtypeStruct((NPAD, 8), _F32),
        ],
    )(psum1, cnts, xp, W1l, W1r, b1l.reshape(1, D))

    (psum2,) = _make_agg(False)(h, src, dst, zeros_d)

    wuv = Wlp.reshape(2, D).T
    buv = jnp.concatenate([blp, jnp.zeros((1,), _F32)]).reshape(1, 2)
    uv = pl.pallas_call(
        _tc_layer2,
        grid=(NPAD // BR,),
        in_specs=[
            pl.BlockSpec((NC, BR, D), lambda i: (0, i, 0)),
            pl.BlockSpec((BR, 8), lambda i: (i, 0)),
            pl.BlockSpec((BR, D), lambda i: (i, 0)),
            wspec, wspec, bspec,
            pl.BlockSpec((D, 2), lambda i: (0, 0)),
            pl.BlockSpec((1, 2), lambda i: (0, 0)),
        ],
        out_specs=pl.BlockSpec((BR, 2), lambda i: (i, 0)),
        out_shape=jax.ShapeDtypeStruct((NPAD, 2), _F32),
    )(psum2, rec, h, W2l, W2r, b2l.reshape(1, D), wuv, buv)

    pairk = pl.kernel(
        _pairs_body,
        out_type=jax.ShapeDtypeStruct((P,), _F32),
        mesh=_mk_mesh(),
        compiler_params=pltpu.CompilerParams(needs_layout_passes=False),
        scratch_types=[
            pltpu.VMEM((NPAD,), _F32),
            pltpu.VMEM((NPAD,), _F32),
            pltpu.VMEM((PPT,), jnp.int32),
            pltpu.VMEM((PPT,), jnp.int32),
            pltpu.VMEM((PPT,), _F32),
        ],
    )
    return pairk(uv[:, 0], uv[:, 1], p0, p1)
